# Initial kernel scaffold; baseline (speedup 1.0000x reference)
import jax
import jax.numpy as jnp
from jax.experimental import pallas as pl


def kernel(input, agents_per_sample, W):
    raise NotImplementedError("write your pallas kernel here")


# speedup vs baseline: 2.9132x; 2.9132x over previous
import functools

import jax
import jax.numpy as jnp
from jax.experimental import pallas as pl


def _emb_kernel(x_ref, aps_ref, w_ref, out_ref, *, row_off):
    xb = x_ref[...]
    rows = jax.lax.broadcasted_iota(jnp.int32, xb.shape, 1)
    x = jnp.sum(jnp.where(rows == row_off, xb, jnp.zeros((), xb.dtype)), axis=1)
    logits = jax.lax.dot_general(
        x,
        w_ref[...],
        dimension_numbers=(((1,), (1,)), ((), ())),
        preferred_element_type=jnp.float32,
    )
    mask = aps_ref[...] > 0
    out_ref[...] = jnp.where(mask, logits, jnp.zeros((), logits.dtype))


def kernel(input, agents_per_sample, W):
    B, A, S, D = input.shape
    V = W.shape[0]
    x3 = input.reshape(B, A * S, D)
    blk = (S - 1) // 8
    row_off = (S - 1) % 8
    aps2 = agents_per_sample.reshape(B, 1)

    return pl.pallas_call(
        functools.partial(_emb_kernel, row_off=row_off),
        out_shape=jax.ShapeDtypeStruct((B, V), input.dtype),
        grid=(1,),
        in_specs=[
            pl.BlockSpec((B, 8, D), lambda i: (0, blk, 0)),
            pl.BlockSpec((B, 1), lambda i: (0, 0)),
            pl.BlockSpec((V, D), lambda i: (0, 0)),
        ],
        out_specs=pl.BlockSpec((B, V), lambda i: (0, 0)),
    )(x3, aps2, W)
